# baseline (speedup 1.0000x reference)
import functools

import jax
import jax.numpy as jnp
from jax import lax
from jax.experimental import pallas as pl
from jax.experimental.pallas import tpu as pltpu
from jax.experimental.pallas import tpu_sc as plsc

_B, _N, _K, _C, _M, _OUT = 1, 100000, 16, 16, 16, 64


_NW = 32
_PER_W = (_N * _K) // _NW
_CH = 1000
_NCH = _PER_W // _CH
_NPAIR = _NCH // 2


def _sc_gather_body(idx_hbm, table_hbm, out_hbm, idx_v, rows_a, rows_b,
                    gsem, ssem_a, ssem_b):
    wid = lax.axis_index("s") * 2 + lax.axis_index("c")
    pltpu.sync_copy(idx_hbm.at[wid], idx_v)
    base = wid * _PER_W

    def gather(j, rows):
        pltpu.async_copy(table_hbm.at[idx_v.at[j]], rows, gsem).wait()

    def start_store(j, rows, ssem):
        pltpu.make_async_copy(rows, out_hbm.at[pl.ds(base + j * _CH, _CH)],
                              ssem).start()

    def wait_store(rows, ssem):
        pltpu.make_async_copy(rows, out_hbm.at[pl.ds(base, _CH)], ssem).wait()

    gather(0, rows_a)
    start_store(0, rows_a, ssem_a)
    gather(1, rows_b)
    start_store(1, rows_b, ssem_b)

    def pair(i, carry):
        wait_store(rows_a, ssem_a)
        gather(2 * i, rows_a)
        start_store(2 * i, rows_a, ssem_a)
        wait_store(rows_b, ssem_b)
        gather(2 * i + 1, rows_b)
        start_store(2 * i + 1, rows_b, ssem_b)
        return carry

    lax.fori_loop(1, _NPAIR, pair, 0)
    wait_store(rows_a, ssem_a)
    wait_store(rows_b, ssem_b)


_sc_gather = functools.partial(
    pl.kernel,
    out_type=jax.ShapeDtypeStruct((_N * _K, _C), jnp.float32),
    mesh=plsc.VectorSubcoreMesh(core_axis_name="c", subcore_axis_name="s"),
    scratch_types=[
        pltpu.VMEM((_NCH, _CH), jnp.int32),
        pltpu.VMEM((_CH, _C), jnp.float32),
        pltpu.VMEM((_CH, _C), jnp.float32),
        pltpu.SemaphoreType.DMA,
        pltpu.SemaphoreType.DMA,
        pltpu.SemaphoreType.DMA,
    ],
    compiler_params=pltpu.CompilerParams(use_tc_tiling_on_sc=False),
)(_sc_gather_body)


_BNL = 256
_GRID = -(-_N // _BNL)


def _tc_body(gt_ref, wt_ref, w2_ref, b_ref, ot_ref):
    halves = []
    for h in range(2):
        accs = [jnp.zeros((8, _BNL), jnp.float32) for _ in range(_C)]
        for k in range(_K):
            gk = gt_ref[0, k * _C:(k + 1) * _C, :]
            wkh = wt_ref[0, k * _M + 8 * h:k * _M + 8 * h + 8, :]
            for c in range(_C):
                accs[c] = accs[c] + gk[c][None, :] * wkh
        halves.append(accs)
    rows = []
    for c in range(_C):
        rows.append(halves[0][c])
        rows.append(halves[1][c])
    p = jnp.concatenate(rows, axis=0)
    out_t = lax.dot_general(w2_ref[...], p, (((1,), (0,)), ((), ())),
                            preferred_element_type=jnp.float32)
    ot_ref[0] = out_t + b_ref[...]


_tc_call = pl.pallas_call(
    _tc_body,
    out_shape=jax.ShapeDtypeStruct((_GRID, _OUT, _BNL), jnp.float32),
    grid=(_GRID,),
    in_specs=[
        pl.BlockSpec((1, _K * _C, _BNL), lambda i: (i, 0, 0)),
        pl.BlockSpec((1, _K * _M, _BNL), lambda i: (i, 0, 0)),
        pl.BlockSpec((_OUT, _K * _C), lambda i: (0, 0)),
        pl.BlockSpec((_OUT, _BNL), lambda i: (0, 0)),
    ],
    out_specs=pl.BlockSpec((1, _OUT, _BNL), lambda i: (i, 0, 0)),
)


def kernel(input_features, neighbor_inds, inverse_neighbors, inverse_k,
           inverse_idx, weightnet, linear_weight, linear_bias):
    table = input_features[0]
    idx = neighbor_inds[0].reshape(_NW, _NCH, _CH)
    gathered = _sc_gather(idx, table)
    np_ = _GRID * _BNL
    g2 = jnp.pad(gathered.reshape(_N, _K * _C), ((0, np_ - _N), (0, 0)))
    w2 = jnp.pad(weightnet[0].reshape(_N, _K * _M), ((0, np_ - _N), (0, 0)))
    g_t = g2.reshape(_GRID, _BNL, _K * _C).transpose(0, 2, 1)
    w_t = w2.reshape(_GRID, _BNL, _K * _M).transpose(0, 2, 1)
    bias_t = jnp.tile(linear_bias[:, None], (1, _BNL))
    out3 = _tc_call(g_t, w_t, linear_weight, bias_t)
    out = out3.transpose(0, 2, 1).reshape(np_, _OUT)[:_N]
    return out[None]


# speedup vs baseline: 1.5149x; 1.5149x over previous
import functools

import jax
import jax.numpy as jnp
from jax import lax
from jax.experimental import pallas as pl
from jax.experimental.pallas import tpu as pltpu
from jax.experimental.pallas import tpu_sc as plsc

_B, _N, _K, _C, _M, _OUT = 1, 100000, 16, 16, 16, 64


_NW = 32
_PER_W = (_N * _K) // _NW
_CH = 1000
_NCH = _PER_W // _CH
_NPAIR = _NCH // 2


def _sc_gather_body(idx_hbm, table_hbm, out_hbm, idx_v, rows_a, rows_b,
                    gsem, ssem_a, ssem_b):
    wid = lax.axis_index("s") * 2 + lax.axis_index("c")
    pltpu.sync_copy(idx_hbm.at[wid], idx_v)
    base = wid * _PER_W

    def gather(j, rows):
        pltpu.async_copy(table_hbm.at[idx_v.at[j]], rows, gsem).wait()

    def start_store(j, rows, ssem):
        pltpu.make_async_copy(rows, out_hbm.at[pl.ds(base + j * _CH, _CH)],
                              ssem).start()

    def wait_store(rows, ssem):
        pltpu.make_async_copy(rows, out_hbm.at[pl.ds(base, _CH)], ssem).wait()

    gather(0, rows_a)
    start_store(0, rows_a, ssem_a)
    gather(1, rows_b)
    start_store(1, rows_b, ssem_b)

    def pair(i, carry):
        wait_store(rows_a, ssem_a)
        gather(2 * i, rows_a)
        start_store(2 * i, rows_a, ssem_a)
        wait_store(rows_b, ssem_b)
        gather(2 * i + 1, rows_b)
        start_store(2 * i + 1, rows_b, ssem_b)
        return carry

    lax.fori_loop(1, _NPAIR, pair, 0)
    wait_store(rows_a, ssem_a)
    wait_store(rows_b, ssem_b)


_sc_gather = functools.partial(
    pl.kernel,
    out_type=jax.ShapeDtypeStruct((_N * _K, _C), jnp.float32),
    mesh=plsc.VectorSubcoreMesh(core_axis_name="c", subcore_axis_name="s"),
    scratch_types=[
        pltpu.VMEM((_NCH, _CH), jnp.int32),
        pltpu.VMEM((_CH, _C), jnp.float32),
        pltpu.VMEM((_CH, _C), jnp.float32),
        pltpu.SemaphoreType.DMA,
        pltpu.SemaphoreType.DMA,
        pltpu.SemaphoreType.DMA,
    ],
    compiler_params=pltpu.CompilerParams(use_tc_tiling_on_sc=False),
)(_sc_gather_body)


_BNL = 512
_GRID = -(-_N // _BNL)


def _tc_body(gt_ref, wt_ref, w2_ref, b_ref, ot_ref):
    quarters = []
    for q in range(4):
        accs = [jnp.zeros((_M, _BNL), jnp.float32) for _ in range(4)]
        for k in range(_K):
            wk = wt_ref[k * _M:(k + 1) * _M, :]
            for cc in range(4):
                grow = gt_ref[k * _C + q * 4 + cc, :]
                accs[cc] = accs[cc] + grow[None, :] * wk
        quarters.extend(accs)
    p = jnp.concatenate(quarters, axis=0)
    out_t = lax.dot_general(w2_ref[...], p, (((1,), (0,)), ((), ())),
                            preferred_element_type=jnp.float32)
    ot_ref[...] = out_t + b_ref[...]


_tc_call = pl.pallas_call(
    _tc_body,
    out_shape=jax.ShapeDtypeStruct((_OUT, _N), jnp.float32),
    grid=(_GRID,),
    in_specs=[
        pl.BlockSpec((_K * _C, _BNL), lambda i: (0, i)),
        pl.BlockSpec((_K * _M, _BNL), lambda i: (0, i)),
        pl.BlockSpec((_OUT, _K * _C), lambda i: (0, 0)),
        pl.BlockSpec((_OUT, _BNL), lambda i: (0, 0)),
    ],
    out_specs=pl.BlockSpec((_OUT, _BNL), lambda i: (0, i)),
)


def kernel(input_features, neighbor_inds, inverse_neighbors, inverse_k,
           inverse_idx, weightnet, linear_weight, linear_bias):
    table = input_features[0]
    idx = neighbor_inds[0].reshape(_NW, _NCH, _CH)
    gathered = _sc_gather(idx, table)
    g_t = gathered.reshape(_N, _K * _C).T
    w_t = weightnet[0].reshape(_N, _K * _M).T
    bias_t = jnp.tile(linear_bias[:, None], (1, _BNL))
    out_t = _tc_call(g_t, w_t, linear_weight, bias_t)
    return out_t.T[None]
